# baseline (speedup 1.0000x reference)
import functools

import jax
import jax.numpy as jnp
from jax import lax
from jax.experimental import pallas as pl
from jax.experimental.pallas import tpu as pltpu
from jax.experimental.pallas import tpu_sc as plsc


def _build_sc_kernel(B, F, N, K, NC, NS):
    NW = NC * NS
    CB = 64
    IPC = CB * F
    assert IPC % 128 == 0
    JG = IPC // 128
    assert B % (NW * CB) == 0
    NCHUNK = B // (NW * CB)
    ROWS_W = B // NW
    NT = IPC // 16

    mesh = plsc.VectorSubcoreMesh(core_axis_name="c", subcore_axis_name="s")

    @functools.partial(
        pl.kernel,
        out_type=jax.ShapeDtypeStruct((B, K), jnp.float32),
        mesh=mesh,
        scratch_types=[
            pltpu.VMEM((ROWS_W, F), jnp.int32),
            pltpu.VMEM((JG, 128), jnp.int32),
            pltpu.VMEM((IPC, K), jnp.float32),
            pltpu.VMEM((IPC + 16,), jnp.float32),
            pltpu.VMEM((CB, K), jnp.float32),
            pltpu.VMEM((16,), jnp.float32),
            pltpu.SemaphoreType.DMA,
        ],
        compiler_params=pltpu.CompilerParams(
            use_tc_tiling_on_sc=False, needs_layout_passes=False),
    )
    def fm_kernel(idx_hbm, w0_hbm, w1_hbm, v_hbm, out_hbm,
                  idx2d, idx_flat, rows_v, w1_v, out_v, w0_v, sem):
        cid = lax.axis_index("c")
        sid = lax.axis_index("s")
        wid = sid * NC + cid
        base_row = wid * ROWS_W

        pltpu.sync_copy(w0_hbm, w0_v)
        pltpu.sync_copy(idx_hbm.at[pl.ds(base_row, ROWS_W), :], idx2d)

        w0vec = w0_v[...]
        lane = lax.iota(jnp.int32, 16)
        fzero = jnp.zeros((16,), jnp.float32)

        def chunk_body(c, carry):
            row0 = base_row + c * CB

            c_row = c * CB
            for t in range(NT):
                p = lane + t * 16
                rvec = c_row + p // F
                cvec = p % F
                vals = plsc.load_gather(idx2d, [rvec, cvec])
                idx_flat[t // 8, pl.ds((t % 8) * 16, 16)] = vals

            copies = []
            for j in range(JG):
                copies.append(pltpu.async_copy(
                    v_hbm.at[idx_flat.at[j]],
                    rows_v.at[pl.ds(j * 128, 128), :], sem))
                copies.append(pltpu.async_copy(
                    w1_hbm.at[idx_flat.at[j]],
                    w1_v.at[pl.ds(j * 128, 128)], sem))
            for cp in copies:
                cp.wait()

            def row_body(b, _):
                rbase = b * F
                x = rows_v[rbase]
                s = x
                sq = x * x
                for f in range(1, F):
                    x = rows_v[rbase + f]
                    s = s + x
                    sq = sq + x * x
                wa = w1_v[pl.ds(rbase, 16)]
                wb = jnp.where(
                    lane < (F - 16), w1_v[pl.ds(rbase + 16, 16)], fzero)
                lin_b = jnp.sum(wa) + jnp.sum(wb)
                out_v[b] = 0.5 * (s * s - sq) + (lin_b + w0vec)
                return 0

            lax.fori_loop(0, CB, row_body, 0, unroll=2)

            pltpu.sync_copy(out_v, out_hbm.at[pl.ds(row0, CB), :])
            return carry

        lax.fori_loop(0, NCHUNK, chunk_body, 0)

    return fm_kernel


def _tc_transpose(vt, N, K):
    BLK = 4096
    nb = (N + BLK - 1) // BLK

    def tbody(vt_ref, o_ref):
        r = lax.broadcasted_iota(jnp.int32, (K, K), 0)
        c = lax.broadcasted_iota(jnp.int32, (K, K), 1)
        eye = jnp.where(r == c, 1.0, 0.0).astype(jnp.float32)
        o_ref[...] = lax.dot_general(
            vt_ref[...], eye, (((0,), (0,)), ((), ())),
            preferred_element_type=jnp.float32)

    return pl.pallas_call(
        tbody,
        grid=(nb,),
        in_specs=[pl.BlockSpec((K, BLK), lambda i: (0, i))],
        out_specs=pl.BlockSpec((BLK, K), lambda i: (i, 0)),
        out_shape=jax.ShapeDtypeStruct((N, K), jnp.float32),
    )(vt)


def kernel(inputs, W0, W1, V):
    B, F = inputs.shape
    N, K = V.shape
    info = plsc.get_sparse_core_info()
    NC, NS = info.num_cores, info.num_subcores
    fm = _build_sc_kernel(B, F, N, K, NC, NS)
    w0_16 = jnp.broadcast_to(W0.astype(jnp.float32), (16,))
    w1_flat = W1.astype(jnp.float32).reshape(N)
    v_rows = _tc_transpose(jnp.transpose(V), N, K)
    return fm(inputs.astype(jnp.int32), w0_16, w1_flat, v_rows)


# speedup vs baseline: 1.1351x; 1.1351x over previous
import functools

import jax
import jax.numpy as jnp
from jax import lax
from jax.experimental import pallas as pl
from jax.experimental.pallas import tpu as pltpu
from jax.experimental.pallas import tpu_sc as plsc


def _build_sc_kernel(B, F, N, K, NC, NS):
    NW = NC * NS
    CB = 64
    IPC = CB * F
    assert IPC % 128 == 0
    JG = IPC // 128
    assert B % (NW * CB) == 0
    NCHUNK = B // (NW * CB)
    ROWS_W = B // NW
    NT = IPC // 16

    mesh = plsc.VectorSubcoreMesh(core_axis_name="c", subcore_axis_name="s")

    @functools.partial(
        pl.kernel,
        out_type=jax.ShapeDtypeStruct((B, K), jnp.float32),
        mesh=mesh,
        scratch_types=[
            pltpu.VMEM((ROWS_W, F), jnp.int32),
            pltpu.VMEM((JG, 128), jnp.int32),
            pltpu.VMEM((JG, 128), jnp.int32),
            pltpu.VMEM((IPC, K), jnp.float32),
            pltpu.VMEM((IPC + 16,), jnp.float32),
            pltpu.VMEM((CB, K), jnp.float32),
            pltpu.VMEM((16,), jnp.float32),
            pltpu.SemaphoreType.DMA,
        ],
        compiler_params=pltpu.CompilerParams(
            use_tc_tiling_on_sc=False, needs_layout_passes=False),
    )
    def fm_kernel(idx_hbm, w0_hbm, w1_hbm, v_hbm, out_hbm,
                  idx2d, idx_flat, idx_perm, rows_v, w1_v, out_v, w0_v, sem):
        cid = lax.axis_index("c")
        sid = lax.axis_index("s")
        wid = sid * NC + cid
        base_row = wid * ROWS_W

        pltpu.sync_copy(w0_hbm, w0_v)
        pltpu.sync_copy(idx_hbm.at[pl.ds(base_row, ROWS_W), :], idx2d)

        w0vec = w0_v[...]
        lane = lax.iota(jnp.int32, 16)
        fzero = jnp.zeros((16,), jnp.float32)

        def chunk_body(c, carry):
            row0 = base_row + c * CB

            c_row = c * CB
            for t in range(NT):
                p = lane + t * 16
                rvec = c_row + p // F
                cvec = p % F
                vals = plsc.load_gather(idx2d, [rvec, cvec])
                idx_flat[t // 8, pl.ds((t % 8) * 16, 16)] = vals
                blk = vals & 4095
                pv = (vals - blk) + ((vals & 511) << 3) + (blk >> 9)
                idx_perm[t // 8, pl.ds((t % 8) * 16, 16)] = pv

            copies = []
            for j in range(JG):
                copies.append(pltpu.async_copy(
                    v_hbm.at[idx_perm.at[j]],
                    rows_v.at[pl.ds(j * 128, 128), :], sem))
                copies.append(pltpu.async_copy(
                    w1_hbm.at[idx_flat.at[j]],
                    w1_v.at[pl.ds(j * 128, 128)], sem))
            for cp in copies:
                cp.wait()

            def row_body(b, _):
                rbase = b * F
                x = rows_v[rbase]
                s = x
                sq = x * x
                for f in range(1, F):
                    x = rows_v[rbase + f]
                    s = s + x
                    sq = sq + x * x
                wa = w1_v[pl.ds(rbase, 16)]
                wb = jnp.where(
                    lane < (F - 16), w1_v[pl.ds(rbase + 16, 16)], fzero)
                lin_b = jnp.sum(wa) + jnp.sum(wb)
                out_v[b] = 0.5 * (s * s - sq) + (lin_b + w0vec)
                return 0

            lax.fori_loop(0, CB, row_body, 0, unroll=2)

            pltpu.sync_copy(out_v, out_hbm.at[pl.ds(row0, CB), :])
            return carry

        lax.fori_loop(0, NCHUNK, chunk_body, 0)

    return fm_kernel


def _tc_transpose(vt, N, K):
    BLK = 4096
    nb = (N + BLK - 1) // BLK
    GR = 128 // K
    OR = BLK // GR

    def tbody(vt_ref, o_ref):
        r = lax.broadcasted_iota(jnp.int32, (K, K), 0)
        c = lax.broadcasted_iota(jnp.int32, (K, K), 1)
        eye = jnp.where(r == c, 1.0, 0.0).astype(jnp.float32)
        for a in range(GR):
            piece = lax.dot_general(
                vt_ref[:, OR * a:OR * (a + 1)], eye, (((0,), (0,)), ((), ())),
                precision=lax.Precision.HIGHEST,
                preferred_element_type=jnp.float32)
            o_ref[:, K * a:K * (a + 1)] = piece

    return pl.pallas_call(
        tbody,
        grid=(nb,),
        in_specs=[pl.BlockSpec((K, BLK), lambda i: (0, i))],
        out_specs=pl.BlockSpec((OR, 128), lambda i: (i, 0)),
        out_shape=jax.ShapeDtypeStruct((nb * OR, 128), jnp.float32),
    )(vt)


def kernel(inputs, W0, W1, V):
    B, F = inputs.shape
    N, K = V.shape
    info = plsc.get_sparse_core_info()
    NC, NS = info.num_cores, info.num_subcores
    fm = _build_sc_kernel(B, F, N, K, NC, NS)
    w0_16 = jnp.broadcast_to(W0.astype(jnp.float32), (16,))
    w1_flat = W1.astype(jnp.float32).reshape(N)
    v_packed = _tc_transpose(jnp.transpose(V), N, K)
    v_rows = v_packed.reshape(v_packed.shape[0] * (128 // K), K)
    return fm(inputs.astype(jnp.int32), w0_16, w1_flat, v_rows)


# speedup vs baseline: 1.1805x; 1.0400x over previous
import functools

import jax
import jax.numpy as jnp
from jax import lax
from jax.experimental import pallas as pl
from jax.experimental.pallas import tpu as pltpu
from jax.experimental.pallas import tpu_sc as plsc


def _build_sc_kernel(B, F, N, K, NC, NS):
    NW = NC * NS
    CB = 64
    IPC = CB * F
    assert IPC % 128 == 0
    JG = IPC // 128
    assert B % (NW * CB) == 0
    NCHUNK = B // (NW * CB)
    ROWS_W = B // NW
    NT = IPC // 16

    mesh = plsc.VectorSubcoreMesh(core_axis_name="c", subcore_axis_name="s")

    @functools.partial(
        pl.kernel,
        out_type=jax.ShapeDtypeStruct((B, K), jnp.float32),
        mesh=mesh,
        scratch_types=[
            pltpu.VMEM((ROWS_W, F), jnp.int32),
            pltpu.VMEM((JG, 128), jnp.int32),
            pltpu.VMEM((JG, 128), jnp.int32),
            pltpu.VMEM((IPC, K), jnp.float32),
            pltpu.VMEM((IPC + 16,), jnp.float32),
            pltpu.VMEM((CB, K), jnp.float32),
            pltpu.VMEM((16,), jnp.float32),
            pltpu.SemaphoreType.DMA,
        ],
        compiler_params=pltpu.CompilerParams(
            use_tc_tiling_on_sc=False, needs_layout_passes=False),
    )
    def fm_kernel(idx_hbm, w0_hbm, w1_hbm, v_hbm, out_hbm,
                  idx2d, idx_flat, idx_perm, rows_v, w1_v, out_v, w0_v, sem):
        cid = lax.axis_index("c")
        sid = lax.axis_index("s")
        wid = sid * NC + cid
        base_row = wid * ROWS_W

        pltpu.sync_copy(w0_hbm, w0_v)
        pltpu.sync_copy(idx_hbm.at[pl.ds(base_row, ROWS_W), :], idx2d)

        w0vec = w0_v[...]
        lane = lax.iota(jnp.int32, 16)
        fzero = jnp.zeros((16,), jnp.float32)

        def chunk_body(c, carry):
            row0 = base_row + c * CB

            c_row = c * CB
            for t in range(NT):
                p = lane + t * 16
                rvec = c_row + p // F
                cvec = p % F
                vals = plsc.load_gather(idx2d, [rvec, cvec])
                idx_flat[t // 8, pl.ds((t % 8) * 16, 16)] = vals
                blk = vals & 4095
                pv = (vals - blk) + ((vals & 511) << 3) + (blk >> 9)
                idx_perm[t // 8, pl.ds((t % 8) * 16, 16)] = pv

            copies = []
            for j in range(JG):
                copies.append(pltpu.async_copy(
                    v_hbm.at[idx_perm.at[j]],
                    rows_v.at[pl.ds(j * 128, 128), :], sem))
                copies.append(pltpu.async_copy(
                    w1_hbm.at[idx_flat.at[j]],
                    w1_v.at[pl.ds(j * 128, 128)], sem))
            for cp in copies:
                cp.wait()

            def row_body(b, _):
                rbase = b * F
                x = rows_v[rbase]
                s = x
                sq = x * x
                for f in range(1, F):
                    x = rows_v[rbase + f]
                    s = s + x
                    sq = sq + x * x
                wa = w1_v[pl.ds(rbase, 16)]
                wb = jnp.where(
                    lane < (F - 16), w1_v[pl.ds(rbase + 16, 16)], fzero)
                lin_b = jnp.sum(wa) + jnp.sum(wb)
                out_v[b] = 0.5 * (s * s - sq) + (lin_b + w0vec)
                return 0

            lax.fori_loop(0, CB, row_body, 0, unroll=2)

            pltpu.sync_copy(out_v, out_hbm.at[pl.ds(row0, CB), :])
            return carry

        lax.fori_loop(0, NCHUNK, chunk_body, 0)

    return fm_kernel


def _tc_transpose(vt, N, K):
    BLK = 4096
    nb = (N + BLK - 1) // BLK
    GR = 128 // K
    OR = BLK // GR

    def tbody(vt_ref, o_ref):
        r = lax.broadcasted_iota(jnp.int32, (K, K), 0)
        c = lax.broadcasted_iota(jnp.int32, (K, K), 1)
        eye = jnp.where(r == c, 1.0, 0.0).astype(jnp.float32)
        t = lax.dot_general(
            vt_ref[...], eye, (((0,), (0,)), ((), ())),
            precision=lax.Precision.HIGHEST,
            preferred_element_type=jnp.float32)
        o_ref[...] = jnp.concatenate(
            [t[OR * a:OR * (a + 1), :] for a in range(GR)], axis=1)

    return pl.pallas_call(
        tbody,
        grid=(nb,),
        in_specs=[pl.BlockSpec((K, BLK), lambda i: (0, i))],
        out_specs=pl.BlockSpec((OR, 128), lambda i: (i, 0)),
        out_shape=jax.ShapeDtypeStruct((nb * OR, 128), jnp.float32),
    )(vt)


def kernel(inputs, W0, W1, V):
    B, F = inputs.shape
    N, K = V.shape
    info = plsc.get_sparse_core_info()
    NC, NS = info.num_cores, info.num_subcores
    fm = _build_sc_kernel(B, F, N, K, NC, NS)
    w0_16 = jnp.broadcast_to(W0.astype(jnp.float32), (16,))
    w1_flat = W1.astype(jnp.float32).reshape(N)
    v_packed = _tc_transpose(jnp.transpose(V), N, K)
    v_rows = v_packed.reshape(v_packed.shape[0] * (128 // K), K)
    return fm(inputs.astype(jnp.int32), w0_16, w1_flat, v_rows)


# speedup vs baseline: 1.7305x; 1.4659x over previous
import functools

import jax
import jax.numpy as jnp
from jax import lax
from jax.experimental import pallas as pl
from jax.experimental.pallas import tpu as pltpu
from jax.experimental.pallas import tpu_sc as plsc


def _build_sc_kernel(B, F, N, K, NC, NS):
    NW = NC * NS
    CB = 64
    IPC = CB * F
    assert IPC % 128 == 0
    JG = IPC // 128
    assert B % (NW * CB) == 0
    NCHUNK = B // (NW * CB)
    ROWS_W = B // NW
    NT = IPC // 16

    mesh = plsc.VectorSubcoreMesh(core_axis_name="c", subcore_axis_name="s")

    @functools.partial(
        pl.kernel,
        out_type=jax.ShapeDtypeStruct((B, K), jnp.float32),
        mesh=mesh,
        scratch_types=[
            pltpu.VMEM((ROWS_W, F), jnp.int32),
            pltpu.VMEM((JG, 128), jnp.int32),
            pltpu.VMEM((JG, 128), jnp.int32),
            pltpu.VMEM((IPC, K), jnp.float32),
            pltpu.VMEM((IPC + 16,), jnp.float32),
            pltpu.VMEM((CB, K), jnp.float32),
            pltpu.VMEM((16,), jnp.float32),
            pltpu.SemaphoreType.DMA,
        ],
        compiler_params=pltpu.CompilerParams(
            use_tc_tiling_on_sc=False, needs_layout_passes=False),
    )
    def fm_kernel(idx_hbm, w0_hbm, w1_hbm, v_hbm, out_hbm,
                  idx2d, idx_flat, idx_perm, rows_v, w1_v, out_v, w0_v, sem):
        cid = lax.axis_index("c")
        sid = lax.axis_index("s")
        wid = sid * NC + cid
        base_row = wid * ROWS_W

        pltpu.sync_copy(w0_hbm, w0_v)
        pltpu.sync_copy(idx_hbm.at[pl.ds(base_row, ROWS_W), :], idx2d)

        w0vec = w0_v[...]
        lane = lax.iota(jnp.int32, 16)
        fzero = jnp.zeros((16,), jnp.float32)

        def chunk_body(c, carry):
            row0 = base_row + c * CB

            c_row = c * CB
            for t in range(NT):
                p = lane + t * 16
                rvec = c_row + p // F
                cvec = p % F
                vals = plsc.load_gather(idx2d, [rvec, cvec])
                idx_flat[t // 8, pl.ds((t % 8) * 16, 16)] = vals
                blk = vals & 4095
                pv = (vals - blk) + ((vals & 511) << 3) + (blk >> 9)
                idx_perm[t // 8, pl.ds((t % 8) * 16, 16)] = pv

            copies = []
            for j in range(JG):
                copies.append(pltpu.async_copy(
                    v_hbm.at[idx_perm.at[j]],
                    rows_v.at[pl.ds(j * 128, 128), :], sem))
                copies.append(pltpu.async_copy(
                    w1_hbm.at[idx_flat.at[j]],
                    w1_v.at[pl.ds(j * 128, 128)], sem))
            for cp in copies:
                cp.wait()

            def row_body(b, _):
                rbase = b * F
                x = rows_v[rbase]
                s = x
                sq = x * x
                for f in range(1, F):
                    x = rows_v[rbase + f]
                    s = s + x
                    sq = sq + x * x
                wa = w1_v[pl.ds(rbase, 16)]
                wb = jnp.where(
                    lane < (F - 16), w1_v[pl.ds(rbase + 16, 16)], fzero)
                lin_b = jnp.sum(wa) + jnp.sum(wb)
                out_v[b] = 0.5 * (s * s - sq) + (lin_b + w0vec)
                return 0

            lax.fori_loop(0, CB, row_body, 0, unroll=2)

            pltpu.sync_copy(out_v, out_hbm.at[pl.ds(row0, CB), :])
            return carry

        lax.fori_loop(0, NCHUNK, chunk_body, 0)

    return fm_kernel


def _tc_transpose(vt, N, K):
    BLK = 4096
    nb = (N + BLK - 1) // BLK
    GR = 128 // K
    OR = BLK // GR

    def tbody(vt_ref, o_ref):
        t = vt_ref[...].T
        o_ref[...] = jnp.concatenate(
            [t[OR * a:OR * (a + 1), :] for a in range(GR)], axis=1)

    return pl.pallas_call(
        tbody,
        grid=(nb,),
        in_specs=[pl.BlockSpec((K, BLK), lambda i: (0, i))],
        out_specs=pl.BlockSpec((OR, 128), lambda i: (i, 0)),
        out_shape=jax.ShapeDtypeStruct((nb * OR, 128), jnp.float32),
    )(vt)


def kernel(inputs, W0, W1, V):
    B, F = inputs.shape
    N, K = V.shape
    info = plsc.get_sparse_core_info()
    NC, NS = info.num_cores, info.num_subcores
    fm = _build_sc_kernel(B, F, N, K, NC, NS)
    w0_16 = jnp.broadcast_to(W0.astype(jnp.float32), (16,))
    w1_flat = W1.astype(jnp.float32).reshape(N)
    v_packed = _tc_transpose(jnp.transpose(V), N, K)
    v_rows = v_packed.reshape(v_packed.shape[0] * (128 // K), K)
    return fm(inputs.astype(jnp.int32), w0_16, w1_flat, v_rows)


# speedup vs baseline: 1.7966x; 1.0382x over previous
import functools

import jax
import jax.numpy as jnp
from jax import lax
from jax.experimental import pallas as pl
from jax.experimental.pallas import tpu as pltpu
from jax.experimental.pallas import tpu_sc as plsc


PACK_BLK = 8192


def _build_sc_kernel(B, F, N, K, NC, NS):
    PM = PACK_BLK - 1
    GR2 = 128 // K
    OM = PACK_BLK // GR2 - 1
    OS = (PACK_BLK // GR2).bit_length() - 1
    NW = NC * NS
    CB = 64
    IPC = CB * F
    assert IPC % 128 == 0
    JG = IPC // 128
    assert B % (NW * CB) == 0
    NCHUNK = B // (NW * CB)
    ROWS_W = B // NW
    NT = IPC // 16

    mesh = plsc.VectorSubcoreMesh(core_axis_name="c", subcore_axis_name="s")

    @functools.partial(
        pl.kernel,
        out_type=jax.ShapeDtypeStruct((B, K), jnp.float32),
        mesh=mesh,
        scratch_types=[
            pltpu.VMEM((ROWS_W, F), jnp.int32),
            pltpu.VMEM((JG, 128), jnp.int32),
            pltpu.VMEM((JG, 128), jnp.int32),
            pltpu.VMEM((IPC, K), jnp.float32),
            pltpu.VMEM((IPC + 16,), jnp.float32),
            pltpu.VMEM((CB, K), jnp.float32),
            pltpu.VMEM((16,), jnp.float32),
            pltpu.SemaphoreType.DMA,
        ],
        compiler_params=pltpu.CompilerParams(
            use_tc_tiling_on_sc=False, needs_layout_passes=False),
    )
    def fm_kernel(idx_hbm, w0_hbm, w1_hbm, v_hbm, out_hbm,
                  idx2d, idx_flat, idx_perm, rows_v, w1_v, out_v, w0_v, sem):
        cid = lax.axis_index("c")
        sid = lax.axis_index("s")
        wid = sid * NC + cid
        base_row = wid * ROWS_W

        pltpu.sync_copy(w0_hbm, w0_v)
        pltpu.sync_copy(idx_hbm.at[pl.ds(base_row, ROWS_W), :], idx2d)

        w0vec = w0_v[...]
        lane = lax.iota(jnp.int32, 16)
        fzero = jnp.zeros((16,), jnp.float32)

        def chunk_body(c, carry):
            row0 = base_row + c * CB

            c_row = c * CB
            for t in range(NT):
                p = lane + t * 16
                rvec = c_row + p // F
                cvec = p % F
                vals = plsc.load_gather(idx2d, [rvec, cvec])
                idx_flat[t // 8, pl.ds((t % 8) * 16, 16)] = vals
                blk = vals & PM
                pv = (vals - blk) + ((vals & OM) << 3) + (blk >> OS)
                idx_perm[t // 8, pl.ds((t % 8) * 16, 16)] = pv

            copies = []
            for j in range(JG):
                copies.append(pltpu.async_copy(
                    v_hbm.at[idx_perm.at[j]],
                    rows_v.at[pl.ds(j * 128, 128), :], sem))
                copies.append(pltpu.async_copy(
                    w1_hbm.at[idx_flat.at[j]],
                    w1_v.at[pl.ds(j * 128, 128)], sem))
            for cp in copies:
                cp.wait()

            def row_body(b, _):
                rbase = b * F
                x = rows_v[rbase]
                s = x
                sq = x * x
                for f in range(1, F):
                    x = rows_v[rbase + f]
                    s = s + x
                    sq = sq + x * x
                wa = w1_v[pl.ds(rbase, 16)]
                wb = jnp.where(
                    lane < (F - 16), w1_v[pl.ds(rbase + 16, 16)], fzero)
                lin_b = jnp.sum(wa) + jnp.sum(wb)
                out_v[b] = 0.5 * (s * s - sq) + (lin_b + w0vec)
                return 0

            lax.fori_loop(0, CB, row_body, 0, unroll=2)

            pltpu.sync_copy(out_v, out_hbm.at[pl.ds(row0, CB), :])
            return carry

        lax.fori_loop(0, NCHUNK, chunk_body, 0)

    return fm_kernel


def _tc_transpose(vt, N, K):
    BLK = PACK_BLK
    nb = (N + BLK - 1) // BLK
    GR = 128 // K
    OR = BLK // GR

    def tbody(vt_ref, o_ref):
        for a in range(GR):
            o_ref[:, K * a:K * (a + 1)] = vt_ref[:, OR * a:OR * (a + 1)].T

    return pl.pallas_call(
        tbody,
        grid=(nb,),
        in_specs=[pl.BlockSpec((K, BLK), lambda i: (0, i))],
        out_specs=pl.BlockSpec((OR, 128), lambda i: (i, 0)),
        out_shape=jax.ShapeDtypeStruct((nb * OR, 128), jnp.float32),
    )(vt)


def kernel(inputs, W0, W1, V):
    B, F = inputs.shape
    N, K = V.shape
    info = plsc.get_sparse_core_info()
    NC, NS = info.num_cores, info.num_subcores
    fm = _build_sc_kernel(B, F, N, K, NC, NS)
    w0_16 = jnp.broadcast_to(W0.astype(jnp.float32), (16,))
    w1_flat = W1.astype(jnp.float32).reshape(N)
    v_packed = _tc_transpose(jnp.transpose(V), N, K)
    v_rows = v_packed.reshape(v_packed.shape[0] * (128 // K), K)
    return fm(inputs.astype(jnp.int32), w0_16, w1_flat, v_rows)
